# Initial kernel scaffold; baseline (speedup 1.0000x reference)
import jax
import jax.numpy as jnp
from jax.experimental import pallas as pl


def kernel(xyz, new_xyz, k):
    raise NotImplementedError("write your pallas kernel here")


# speedup vs baseline: 8.0672x; 8.0672x over previous
import functools
import math

import jax
import jax.numpy as jnp
from jax.experimental import pallas as pl
from jax.experimental.pallas import tpu as pltpu

_KSEL = 32
_MT = 128


def _knn_body(xyzt_ref, q_ref, idx_ref, cx_ref, cy_ref, cz_ref,
              rho_ref, th_ref, ph_ref):
    kt = xyzt_ref[0]
    q = q_ref[0]
    n = kt.shape[1]

    q2 = jnp.sum(q * q, axis=1, keepdims=True)
    k2 = jnp.sum(kt * kt, axis=0, keepdims=True)
    cross = jax.lax.dot_general(
        q, kt, (((1,), (0,)), ((), ())),
        preferred_element_type=jnp.float32)
    dist = q2 - 2.0 * cross + k2

    ii = jax.lax.broadcasted_iota(jnp.int32, (_MT, n), 1)
    inf = jnp.float32(jnp.inf)
    big = jnp.int32(n)

    idx_cols = []
    for _ in range(_KSEL):
        m = jnp.min(dist, axis=1, keepdims=True)
        at_min = dist == m
        idx_j = jnp.min(jnp.where(at_min, ii, big), axis=1,
                        keepdims=True)
        idx_cols.append(idx_j)
        dist = jnp.where(ii == idx_j, inf, dist)
    idx_all = jnp.concatenate(idx_cols, axis=1)
    idx_ref[0] = idx_all

    hi = jax.lax.shift_right_logical(idx_all, 7)
    lo = jnp.bitwise_and(idx_all, 127)
    gx = jnp.zeros((_MT, _KSEL), jnp.float32)
    gy = jnp.zeros((_MT, _KSEL), jnp.float32)
    gz = jnp.zeros((_MT, _KSEL), jnp.float32)
    for c in range(n // 128):
        sel = hi == c
        lo_c = jnp.where(sel, lo, 0)
        sx = jnp.broadcast_to(kt[0:1, c * 128:(c + 1) * 128], (_MT, 128))
        sy = jnp.broadcast_to(kt[1:2, c * 128:(c + 1) * 128], (_MT, 128))
        sz = jnp.broadcast_to(kt[2:3, c * 128:(c + 1) * 128], (_MT, 128))
        gx = jnp.where(sel, jnp.take_along_axis(sx, lo_c, axis=1), gx)
        gy = jnp.where(sel, jnp.take_along_axis(sy, lo_c, axis=1), gy)
        gz = jnp.where(sel, jnp.take_along_axis(sz, lo_c, axis=1), gz)

    cx = gx - q[:, 0:1]
    cy = gy - q[:, 1:2]
    cz = gz - q[:, 2:3]

    rho = jnp.sqrt(cx * cx + cy * cy + cz * cz)
    safe_rho = jnp.where(rho == 0.0, 1.0, rho)
    t = jnp.clip(cz / safe_rho, -1.0, 1.0)
    theta = jnp.arctan2(jnp.sqrt(jnp.maximum(1.0 - t * t, 0.0)), t)
    theta = jnp.where(rho == 0.0, 0.0, theta) * jnp.float32(1.0 / math.pi)
    phi = (jnp.arctan2(cy, cx) * jnp.float32(1.0 / (2.0 * math.pi))
           + jnp.float32(0.5))

    cx_ref[0] = cx
    cy_ref[0] = cy
    cz_ref[0] = cz
    rho_ref[0] = rho
    th_ref[0] = theta
    ph_ref[0] = phi


@functools.partial(jax.jit, static_argnames=())
def _knn_pallas(xyz_t, new_xyz):
    b, _, n = xyz_t.shape
    m = new_xyz.shape[1]
    grid = (b, m // _MT)
    ch = jax.ShapeDtypeStruct((b, m, _KSEL), jnp.float32)
    out_shapes = (jax.ShapeDtypeStruct((b, m, _KSEL), jnp.int32),
                  ch, ch, ch, ch, ch, ch)
    out_spec = pl.BlockSpec((1, _MT, _KSEL), lambda i, j: (i, j, 0))
    return pl.pallas_call(
        _knn_body,
        grid=grid,
        in_specs=[
            pl.BlockSpec((1, 3, n), lambda i, j: (i, 0, 0)),
            pl.BlockSpec((1, _MT, 3), lambda i, j: (i, j, 0)),
        ],
        out_specs=(out_spec,) * 7,
        out_shape=out_shapes,
        compiler_params=pltpu.CompilerParams(
            dimension_semantics=("parallel", "arbitrary")),
    )(xyz_t, new_xyz)


def kernel(xyz, new_xyz, k):
    xyz_t = jnp.swapaxes(xyz, 1, 2)
    idx, cx, cy, cz, rho, th, ph = _knn_pallas(xyz_t, new_xyz)
    idx = idx + (jnp.asarray(k, dtype=idx.dtype) - _KSEL)
    feat = jnp.stack([cx, cy, cz, rho, th, ph], axis=-1)
    return feat, idx
